# baseline (speedup 1.0000x reference)
import functools

import jax
import jax.numpy as jnp
from jax import lax
from jax.experimental import pallas as pl
from jax.experimental.pallas import tpu as pltpu
from jax.experimental.pallas import tpu_sc as plsc

N_NODES = 10000
D_IN = 128
D_OUT = 128

NC = 2
NS = 16
NW = NC * NS

CHUNK = 128
AUG = 160
NPAD = 10016
ROWS_PER_TILE = NPAD // NS

N0 = 98
N1 = 60
PADC = 17 * N0 + 15 * N1


def _sc_aggregate():
    mesh = plsc.VectorSubcoreMesh(core_axis_name="c", subcore_axis_name="s")

    @functools.partial(
        pl.kernel,
        out_type=jax.ShapeDtypeStruct((NC, NPAD, AUG), jnp.bfloat16),
        mesh=mesh,
        compiler_params=pltpu.CompilerParams(use_tc_tiling_on_sc=False),
        scratch_types=[
            pltpu.VMEM((N0, 2, CHUNK), jnp.int32),
            pltpu.VMEM((CHUNK, AUG), jnp.bfloat16),
            pltpu.VMEM_SHARED((NPAD, AUG), jnp.bfloat16),
            pltpu.SemaphoreType.DMA,
        ],
    )
    def sc_agg(h_aug, idx4, zeros, out, idx_v, rows, acc, sem):
        cid = lax.axis_index("c")
        sid = lax.axis_index("s")
        r0 = sid * ROWS_PER_TILE
        on0 = cid == 0
        base = jnp.where(on0, sid * N0, 16 * N0 + sid * N1)
        cnt = jnp.where(on0, N0, N1)

        pltpu.sync_copy(idx4.at[pl.ds(base, N0)], idx_v)
        pltpu.async_copy(h_aug.at[idx_v.at[0, 0]], rows, sem)
        pltpu.sync_copy(zeros.at[pl.ds(r0, ROWS_PER_TILE)],
                        acc.at[pl.ds(r0, ROWS_PER_TILE)])
        plsc.subcore_barrier()

        def body(c, carry):
            pltpu.make_async_copy(h_aug.at[idx_v.at[c, 0]], rows, sem).wait()
            pltpu.sync_copy(rows, acc.at[idx_v.at[c, 1]], add=True)

            @pl.when(c + 1 < cnt)
            def _():
                pltpu.async_copy(h_aug.at[idx_v.at[c + 1, 0]], rows, sem)
            return carry

        lax.fori_loop(0, cnt, body, 0)

        plsc.subcore_barrier()
        pltpu.sync_copy(acc.at[pl.ds(r0, ROWS_PER_TILE)],
                        out.at[cid, pl.ds(r0, ROWS_PER_TILE)])

    return sc_agg


def _tc_combine(h_blk, parts_blk, w_blk, b_blk, out_blk):
    p = (parts_blk[0].astype(jnp.float32)
         + parts_blk[1].astype(jnp.float32))
    s = p[:, :D_IN]
    deg = p[:, D_IN:D_IN + 1]
    agg = s / jnp.maximum(deg, 1.0)
    out_blk[...] = (
        jnp.dot(h_blk[...], w_blk[:D_IN], preferred_element_type=jnp.float32)
        + jnp.dot(agg, w_blk[D_IN:], preferred_element_type=jnp.float32)
        + b_blk[...]
    )


def kernel(h, edge_index, W, b):
    src = edge_index[0].astype(jnp.int32)
    dst = edge_index[1].astype(jnp.int32)
    n_edges = src.shape[0]

    e_pad = PADC * CHUNK
    src = jnp.concatenate([src, jnp.zeros((e_pad - n_edges,), jnp.int32)])
    dst = jnp.concatenate(
        [dst, jnp.full((e_pad - n_edges,), N_NODES, jnp.int32)])
    idx4 = jnp.stack([src.reshape(PADC, CHUNK),
                      dst.reshape(PADC, CHUNK)], axis=1)

    h_aug = jnp.concatenate(
        [h, jnp.ones((N_NODES, 1), h.dtype),
         jnp.zeros((N_NODES, AUG - D_IN - 1), h.dtype)],
        axis=1).astype(jnp.bfloat16)
    zeros = jnp.zeros((NPAD, AUG), jnp.bfloat16)

    parts = _sc_aggregate()(h_aug, idx4, zeros)

    blk = 1000
    grid = N_NODES // blk
    out = pl.pallas_call(
        _tc_combine,
        grid=(grid,),
        in_specs=[
            pl.BlockSpec((blk, D_IN), lambda i: (i, 0)),
            pl.BlockSpec((NC, blk, AUG), lambda i: (0, i, 0)),
            pl.BlockSpec((2 * D_IN, D_OUT), lambda i: (0, 0)),
            pl.BlockSpec((1, D_OUT), lambda i: (0, 0)),
        ],
        out_specs=pl.BlockSpec((blk, D_OUT), lambda i: (i, 0)),
        out_shape=jax.ShapeDtypeStruct((N_NODES, D_OUT), jnp.float32),
    )(h, parts, W, b.reshape(1, D_OUT))
    return out


# speedup vs baseline: 1.1914x; 1.1914x over previous
import functools

import jax
import jax.numpy as jnp
from jax import lax
from jax.experimental import pallas as pl
from jax.experimental.pallas import tpu as pltpu
from jax.experimental.pallas import tpu_sc as plsc

N_NODES = 10000
D_IN = 128
D_OUT = 128

NC = 2
NS = 16
NW = NC * NS

CHUNK = 128
AUG = 160
NPAD = 10016
ROWS_PER_TILE = NPAD // NS

N0 = 98
N1 = 59
PADC = 17 * N0 + 15 * N1


def _sc_aggregate():
    mesh = plsc.VectorSubcoreMesh(core_axis_name="c", subcore_axis_name="s")

    @functools.partial(
        pl.kernel,
        out_type=jax.ShapeDtypeStruct((NC, NPAD, AUG), jnp.bfloat16),
        mesh=mesh,
        compiler_params=pltpu.CompilerParams(use_tc_tiling_on_sc=False),
        scratch_types=[
            pltpu.VMEM((N0, 2, CHUNK), jnp.int32),
            pltpu.VMEM((CHUNK, AUG), jnp.bfloat16),
            pltpu.VMEM_SHARED((NPAD, AUG), jnp.bfloat16),
            pltpu.SemaphoreType.DMA,
        ],
    )
    def sc_agg(h_aug, idx4, zeros, out, idx_v, rows, acc, sem):
        cid = lax.axis_index("c")
        sid = lax.axis_index("s")
        r0 = sid * ROWS_PER_TILE
        on0 = cid == 0
        base = jnp.where(on0, sid * N0, 16 * N0 + sid * N1)
        cnt = jnp.where(on0, N0, N1)

        pltpu.sync_copy(zeros.at[pl.ds(r0, ROWS_PER_TILE)],
                        acc.at[pl.ds(r0, ROWS_PER_TILE)])
        pltpu.sync_copy(idx4.at[pl.ds(base, N0)], idx_v)
        plsc.subcore_barrier()

        def body(c, carry):
            pltpu.async_copy(h_aug.at[idx_v.at[c, 0]], rows, sem).wait()
            pltpu.sync_copy(rows, acc.at[idx_v.at[c, 1]], add=True)
            return carry

        lax.fori_loop(0, cnt, body, 0)

        plsc.subcore_barrier()
        pltpu.sync_copy(acc.at[pl.ds(r0, ROWS_PER_TILE)],
                        out.at[cid, pl.ds(r0, ROWS_PER_TILE)])

    return sc_agg


def _tc_combine(h_blk, parts_blk, w_blk, b_blk, out_blk):
    p = (parts_blk[0].astype(jnp.float32)
         + parts_blk[1].astype(jnp.float32))
    s = p[:, :D_IN]
    deg = p[:, D_IN:D_IN + 1]
    agg = s / jnp.maximum(deg, 1.0)
    out_blk[...] = (
        jnp.dot(h_blk[...], w_blk[:D_IN], preferred_element_type=jnp.float32)
        + jnp.dot(agg, w_blk[D_IN:], preferred_element_type=jnp.float32)
        + b_blk[...]
    )


def kernel(h, edge_index, W, b):
    src = edge_index[0].astype(jnp.int32)
    dst = edge_index[1].astype(jnp.int32)
    n_edges = src.shape[0]

    e_pad = PADC * CHUNK
    src = jnp.concatenate([src, jnp.zeros((e_pad - n_edges,), jnp.int32)])
    dst = jnp.concatenate(
        [dst, jnp.full((e_pad - n_edges,), N_NODES, jnp.int32)])
    idx4 = jnp.stack([src.reshape(PADC, CHUNK),
                      dst.reshape(PADC, CHUNK)], axis=1)

    h_aug = jnp.concatenate(
        [h, jnp.ones((N_NODES, 1), h.dtype),
         jnp.zeros((N_NODES, AUG - D_IN - 1), h.dtype)],
        axis=1).astype(jnp.bfloat16)
    zeros = jnp.zeros((NPAD, AUG), jnp.bfloat16)

    parts = _sc_aggregate()(h_aug, idx4, zeros)

    blk = 1000
    grid = N_NODES // blk
    out = pl.pallas_call(
        _tc_combine,
        grid=(grid,),
        in_specs=[
            pl.BlockSpec((blk, D_IN), lambda i: (i, 0)),
            pl.BlockSpec((NC, blk, AUG), lambda i: (0, i, 0)),
            pl.BlockSpec((2 * D_IN, D_OUT), lambda i: (0, 0)),
            pl.BlockSpec((1, D_OUT), lambda i: (0, 0)),
        ],
        out_specs=pl.BlockSpec((blk, D_OUT), lambda i: (i, 0)),
        out_shape=jax.ShapeDtypeStruct((N_NODES, D_OUT), jnp.float32),
    )(h, parts, W, b.reshape(1, D_OUT))
    return out


# speedup vs baseline: 1.2886x; 1.0816x over previous
import functools

import jax
import jax.numpy as jnp
from jax import lax
from jax.experimental import pallas as pl
from jax.experimental.pallas import tpu as pltpu
from jax.experimental.pallas import tpu_sc as plsc

N_NODES = 10000
D_IN = 128
D_OUT = 128

NC = 2
NS = 16
NW = NC * NS

CHUNK = 128
AUG = 160
NPAD = 10016
ROWS_PER_TILE = NPAD // NS

NCHUNKS = 2500
CNT_MAX = 94
SC0_TOTAL = 8 * 94 + 8 * 93


def _sc_aggregate():
    mesh = plsc.VectorSubcoreMesh(core_axis_name="c", subcore_axis_name="s")

    @functools.partial(
        pl.kernel,
        out_type=jax.ShapeDtypeStruct((NC, NPAD, AUG), jnp.bfloat16),
        mesh=mesh,
        compiler_params=pltpu.CompilerParams(use_tc_tiling_on_sc=False),
        scratch_types=[
            pltpu.VMEM((CNT_MAX, CHUNK), jnp.int32),
            pltpu.VMEM((CNT_MAX, CHUNK), jnp.int32),
            pltpu.VMEM((CHUNK, AUG), jnp.bfloat16),
            pltpu.VMEM_SHARED((NPAD, AUG), jnp.bfloat16),
            pltpu.SemaphoreType.DMA,
        ],
    )
    def sc_agg(h_aug, src2, dst2, zeros, out, src_v, dst_v, rows, acc, sem):
        cid = lax.axis_index("c")
        sid = lax.axis_index("s")
        r0 = sid * ROWS_PER_TILE
        on0 = cid == 0
        base = jnp.where(on0,
                         93 * sid + jnp.minimum(sid, 8),
                         SC0_TOTAL + 62 * sid + jnp.minimum(sid, 12))
        cnt = jnp.where(on0,
                        jnp.where(sid < 8, 94, 93),
                        jnp.where(sid < 12, 63, 62))
        sb = jnp.minimum(base, NCHUNKS - CNT_MAX)
        j = base - sb

        pltpu.sync_copy(zeros.at[pl.ds(r0, ROWS_PER_TILE)],
                        acc.at[pl.ds(r0, ROWS_PER_TILE)])
        pltpu.sync_copy(src2.at[pl.ds(sb, CNT_MAX)], src_v)
        pltpu.sync_copy(dst2.at[pl.ds(sb, CNT_MAX)], dst_v)
        plsc.subcore_barrier()

        def body(c, carry):
            pltpu.async_copy(h_aug.at[src_v.at[j + c]], rows, sem).wait()
            pltpu.sync_copy(rows, acc.at[dst_v.at[j + c]], add=True)
            return carry

        lax.fori_loop(0, cnt, body, 0)

        plsc.subcore_barrier()
        pltpu.sync_copy(acc.at[pl.ds(r0, ROWS_PER_TILE)],
                        out.at[cid, pl.ds(r0, ROWS_PER_TILE)])

    return sc_agg


def _tc_combine(h_blk, parts_blk, w_blk, b_blk, out_blk):
    p = (parts_blk[0].astype(jnp.float32)
         + parts_blk[1].astype(jnp.float32))
    s = p[:, :D_IN]
    deg = p[:, D_IN:D_IN + 1]
    agg = s / jnp.maximum(deg, 1.0)
    out_blk[...] = (
        jnp.dot(h_blk[...], w_blk[:D_IN], preferred_element_type=jnp.float32)
        + jnp.dot(agg, w_blk[D_IN:], preferred_element_type=jnp.float32)
        + b_blk[...]
    )


def kernel(h, edge_index, W, b):
    e32 = edge_index.astype(jnp.int32)
    src2 = e32[0].reshape(NCHUNKS, CHUNK)
    dst2 = e32[1].reshape(NCHUNKS, CHUNK)

    h_aug = jnp.concatenate(
        [h, jnp.ones((N_NODES, 1), h.dtype),
         jnp.zeros((N_NODES, AUG - D_IN - 1), h.dtype)],
        axis=1).astype(jnp.bfloat16)
    zeros = jnp.zeros((NPAD, AUG), jnp.bfloat16)

    parts = _sc_aggregate()(h_aug, src2, dst2, zeros)

    blk = 1000
    grid = N_NODES // blk
    out = pl.pallas_call(
        _tc_combine,
        grid=(grid,),
        in_specs=[
            pl.BlockSpec((blk, D_IN), lambda i: (i, 0)),
            pl.BlockSpec((NC, blk, AUG), lambda i: (0, i, 0)),
            pl.BlockSpec((2 * D_IN, D_OUT), lambda i: (0, 0)),
            pl.BlockSpec((1, D_OUT), lambda i: (0, 0)),
        ],
        out_specs=pl.BlockSpec((blk, D_OUT), lambda i: (i, 0)),
        out_shape=jax.ShapeDtypeStruct((N_NODES, D_OUT), jnp.float32),
    )(h, parts, W, b.reshape(1, D_OUT))
    return out
